# baseline (speedup 1.0000x reference)
import functools

import jax
import jax.numpy as jnp
from jax import lax
from jax.experimental import pallas as pl
from jax.experimental.pallas import tpu as pltpu
from jax.experimental.pallas import tpu_sc as plsc

B = 4096
HIST = 50
HIST_PAD = 56
EMB = 64
NW = 32
PW = B // NW
GROUP = 8
NGROUPS = PW // GROUP


def _sc_gather_kernel(wv_hbm, lev_hbm, loc_hbm, samp_hbm,
                      tv_hbm, tl_hbm, ts_hbm,
                      avg_out, lev_out, loc_out, samp_out,
                      idxw, buf0, buf1, outv, sidx, sdst,
                      sem0, sem1, sem2):
    cid = lax.axis_index("c")
    sid = lax.axis_index("s")
    wid = sid * 2 + cid
    base = wid * PW

    pltpu.sync_copy(wv_hbm.at[pl.ds(base * HIST_PAD, PW * HIST_PAD)], idxw)

    def fire(g, buf, sem):
        pltpu.async_copy(
            tv_hbm.at[idxw.at[pl.ds(g * GROUP * HIST_PAD, GROUP * HIST_PAD)]],
            buf, sem)

    def drain(buf, sem):
        pltpu.make_async_copy(tv_hbm.at[pl.ds(0, GROUP * HIST_PAD)], buf, sem).wait()

    def accum(g, buf):
        def rbody(r, _):
            row = g * GROUP + r
            off = r * HIST_PAD
            accs = [buf[off + p, pl.ds(c * 16, 16)]
                    for c in range(EMB // 16) for p in range(2)]
            for jj in range(1, HIST // 2):
                for c in range(EMB // 16):
                    for p in range(2):
                        k = c * 2 + p
                        accs[k] = accs[k] + buf[off + 2 * jj + p,
                                                pl.ds(c * 16, 16)]
            for c in range(EMB // 16):
                outv[row, pl.ds(c * 16, 16)] = (
                    (accs[2 * c] + accs[2 * c + 1]) * jnp.float32(1.0 / HIST))
            return 0
        lax.fori_loop(0, GROUP, rbody, 0)

    fire(0, buf0, sem0)

    for idx_hbm, tab_hbm, out_hbm in ((lev_hbm, tl_hbm, lev_out),
                                      (loc_hbm, tl_hbm, loc_out),
                                      (samp_hbm, ts_hbm, samp_out)):
        pltpu.sync_copy(idx_hbm.at[pl.ds(base, PW)], sidx)
        pltpu.async_copy(tab_hbm.at[sidx], sdst, sem2).wait()
        pltpu.sync_copy(sdst, out_hbm.at[pl.ds(base, PW)])

    def hbody(hg, _):
        g0 = 2 * hg
        g1 = 2 * hg + 1
        fire(g1, buf1, sem1)
        drain(buf0, sem0)
        accum(g0, buf0)

        @pl.when(g1 + 1 < NGROUPS)
        def _():
            fire(g1 + 1, buf0, sem0)

        drain(buf1, sem1)
        accum(g1, buf1)
        return 0

    lax.fori_loop(0, NGROUPS // 2, hbody, 0)

    pltpu.sync_copy(outv, avg_out.at[pl.ds(base, PW)])


def _run_sc_gathers(wv_pad, lev, loc, samp, table_video, table_location, table_sample):
    mesh = plsc.VectorSubcoreMesh(core_axis_name="c", subcore_axis_name="s",
                                  num_cores=2, num_subcores=16)
    f32 = jnp.float32
    out_type = [jax.ShapeDtypeStruct((B, EMB), f32) for _ in range(4)]
    scratch = [
        pltpu.VMEM((PW * HIST_PAD,), jnp.int32),
        pltpu.VMEM((GROUP * HIST_PAD, EMB), f32),
        pltpu.VMEM((GROUP * HIST_PAD, EMB), f32),
        pltpu.VMEM((PW, EMB), f32),
        pltpu.VMEM((PW,), jnp.int32),
        pltpu.VMEM((PW, EMB), f32),
        pltpu.SemaphoreType.DMA,
        pltpu.SemaphoreType.DMA,
        pltpu.SemaphoreType.DMA,
    ]
    run = pl.kernel(_sc_gather_kernel, out_type=out_type, mesh=mesh,
                    scratch_types=scratch,
                    compiler_params=pltpu.CompilerParams(use_tc_tiling_on_sc=False))
    return run(wv_pad, lev, loc, samp, table_video, table_location, table_sample)


def _tc_mean_kernel(esk_ref, ecw_ref, kw_ref, cw_ref):
    kw_ref[...] = jnp.mean(esk_ref[...], axis=1)
    cw_ref[...] = jnp.mean(ecw_ref[...], axis=1)


def _run_tc_means(esk, ecw):
    nb = 8
    blk = B // nb
    f32 = jnp.float32
    return pl.pallas_call(
        _tc_mean_kernel,
        grid=(nb,),
        in_specs=[pl.BlockSpec((blk, 20, EMB), lambda i: (i, 0, 0)),
                  pl.BlockSpec((blk, 20, EMB), lambda i: (i, 0, 0))],
        out_specs=[pl.BlockSpec((blk, EMB), lambda i: (i, 0)),
                   pl.BlockSpec((blk, EMB), lambda i: (i, 0))],
        out_shape=[jax.ShapeDtypeStruct((B, EMB), f32),
                   jax.ShapeDtypeStruct((B, EMB), f32)],
    )(esk, ecw)


def _tc_tower_kernel(avg_ref, kw_ref, cw_ref, lev_ref, loc_ref, misc_ref,
                     bn1g_ref, bn1b_ref, wf_ref, bf_ref, bn2g_ref, bn2b_ref,
                     w1_ref, b1_ref, w2_ref, b2_ref, w3_ref, b3_ref,
                     w4_ref, b4_ref, out_ref):
    uf = jnp.concatenate([avg_ref[...], kw_ref[...], cw_ref[...],
                          lev_ref[...], loc_ref[...], misc_ref[...]], axis=1)

    def bn(x, g, b):
        mu = jnp.mean(x, axis=0)
        xc = x - mu
        var = jnp.mean(xc * xc, axis=0)
        return g * xc / jnp.sqrt(var + 1e-5) + b

    h = bn(uf, bn1g_ref[...], bn1b_ref[...])
    h = jnp.maximum(jnp.dot(h, wf_ref[...]) + bf_ref[...], 0.0)
    h = bn(h, bn2g_ref[...], bn2b_ref[...])
    h = jnp.maximum(jnp.dot(h, w1_ref[...]) + b1_ref[...], 0.0)
    h = jnp.maximum(jnp.dot(h, w2_ref[...]) + b2_ref[...], 0.0)
    h = jnp.maximum(jnp.dot(h, w3_ref[...]) + b3_ref[...], 0.0)
    h = jnp.maximum(jnp.dot(h, w4_ref[...]) + b4_ref[...], 0.0)
    out_ref[...] = h


def _run_tc_tower(avg_video, kw_mean, cw_mean, lev_e, loc_e, misc,
                  bn1_g, bn1_b, WfT, b_fus, bn2_g, bn2_b,
                  W1T, b1, W2T, b2, W3T, b3, W4T, b4):
    f32 = jnp.float32
    args = (avg_video, kw_mean, cw_mean, lev_e, loc_e, misc,
            bn1_g, bn1_b, WfT, b_fus, bn2_g, bn2_b,
            W1T, b1, W2T, b2, W3T, b3, W4T, b4)
    return pl.pallas_call(
        _tc_tower_kernel,
        out_shape=jax.ShapeDtypeStruct((B, 64), f32),
    )(*args)


def kernel(watched_videos, embed_searched_keywords, embed_collect_words,
           level, location, gender, age, platform, example_age, samples,
           table_video, table_sample, table_location,
           bn1_g, bn1_b, W_fus, b_fus, bn2_g, bn2_b,
           W1, b1, W2, b2, W3, b3, W4, b4):
    i32 = jnp.int32
    wv = watched_videos.astype(i32)
    wv_pad = jnp.concatenate(
        [wv, jnp.zeros((B, HIST_PAD - HIST), i32)], axis=1).reshape(B * HIST_PAD)
    lev = level.astype(i32).reshape(B)
    loc = location.astype(i32).reshape(B)
    samp = samples.astype(i32)

    avg_video, lev_e, loc_e, samp_e = _run_sc_gathers(
        wv_pad, lev, loc, samp, table_video, table_location, table_sample)

    kw_mean, cw_mean = _run_tc_means(embed_searched_keywords, embed_collect_words)

    misc = jnp.concatenate(
        [gender, age, platform, example_age, example_age * example_age], axis=1)

    h = _run_tc_tower(avg_video, kw_mean, cw_mean, lev_e, loc_e, misc,
                      bn1_g, bn1_b, W_fus.T, b_fus, bn2_g, bn2_b,
                      W1.T, b1, W2.T, b2, W3.T, b3, W4.T, b4)
    return (h, samp_e)


# speedup vs baseline: 2.7077x; 2.7077x over previous
import functools

import jax
import jax.numpy as jnp
from jax import lax
from jax.experimental import pallas as pl
from jax.experimental.pallas import tpu as pltpu
from jax.experimental.pallas import tpu_sc as plsc

B = 4096
HIST = 50
EMB = 64
NW = 32
PW = B // NW
GROUP = 8
NGROUPS = PW // GROUP


def _sc_gather_kernel(wv_hbm, lev_hbm, loc_hbm, samp_hbm,
                      tv_hbm, tl_hbm, ts_hbm,
                      avg_out, lev_out, loc_out, samp_out,
                      idxw, buf0, buf1, outv, sidx, sdst,
                      sem0, sem1, sem2):
    cid = lax.axis_index("c")
    sid = lax.axis_index("s")
    wid = sid * 2 + cid
    base = wid * PW

    pltpu.sync_copy(wv_hbm.at[pl.ds(base * HIST, PW * HIST)], idxw)

    def fire(g, buf, sem):
        pltpu.async_copy(
            tv_hbm.at[idxw.at[pl.ds(g * GROUP * HIST, GROUP * HIST)]],
            buf, sem)

    def drain(buf, sem):
        pltpu.make_async_copy(tv_hbm.at[pl.ds(0, GROUP * HIST)], buf, sem).wait()

    def accum(g, buf):
        def rbody(r, _):
            row = g * GROUP + r
            off = r * HIST
            accs = [buf[off + p, pl.ds(c * 16, 16)]
                    for c in range(EMB // 16) for p in range(2)]
            for jj in range(1, HIST // 2):
                for c in range(EMB // 16):
                    for p in range(2):
                        k = c * 2 + p
                        accs[k] = accs[k] + buf[off + 2 * jj + p,
                                                pl.ds(c * 16, 16)]
            for c in range(EMB // 16):
                outv[row, pl.ds(c * 16, 16)] = (
                    (accs[2 * c] + accs[2 * c + 1]) * jnp.float32(1.0 / HIST))
            return 0
        lax.fori_loop(0, GROUP, rbody, 0)

    fire(0, buf0, sem0)

    for idx_hbm, tab_hbm, out_hbm in ((lev_hbm, tl_hbm, lev_out),
                                      (loc_hbm, tl_hbm, loc_out),
                                      (samp_hbm, ts_hbm, samp_out)):
        pltpu.sync_copy(idx_hbm.at[pl.ds(base, PW)], sidx)
        pltpu.async_copy(tab_hbm.at[sidx], sdst, sem2).wait()
        pltpu.sync_copy(sdst, out_hbm.at[pl.ds(base, PW)])

    def hbody(hg, _):
        g0 = 2 * hg
        g1 = 2 * hg + 1
        fire(g1, buf1, sem1)
        drain(buf0, sem0)
        accum(g0, buf0)

        @pl.when(g1 + 1 < NGROUPS)
        def _():
            fire(g1 + 1, buf0, sem0)

        drain(buf1, sem1)
        accum(g1, buf1)
        return 0

    lax.fori_loop(0, NGROUPS // 2, hbody, 0)

    pltpu.sync_copy(outv, avg_out.at[pl.ds(base, PW)])


def _run_sc_gathers(wv_pad, lev, loc, samp, table_video, table_location, table_sample):
    mesh = plsc.VectorSubcoreMesh(core_axis_name="c", subcore_axis_name="s",
                                  num_cores=2, num_subcores=16)
    f32 = jnp.float32
    out_type = [jax.ShapeDtypeStruct((B, EMB), f32) for _ in range(4)]
    scratch = [
        pltpu.VMEM((PW * HIST,), jnp.int32),
        pltpu.VMEM((GROUP * HIST, EMB), f32),
        pltpu.VMEM((GROUP * HIST, EMB), f32),
        pltpu.VMEM((PW, EMB), f32),
        pltpu.VMEM((PW,), jnp.int32),
        pltpu.VMEM((PW, EMB), f32),
        pltpu.SemaphoreType.DMA,
        pltpu.SemaphoreType.DMA,
        pltpu.SemaphoreType.DMA,
    ]
    run = pl.kernel(_sc_gather_kernel, out_type=out_type, mesh=mesh,
                    scratch_types=scratch,
                    compiler_params=pltpu.CompilerParams(use_tc_tiling_on_sc=False))
    return run(wv_pad, lev, loc, samp, table_video, table_location, table_sample)


def _tc_mean_kernel(esk_ref, ecw_ref, kw_ref, cw_ref):
    kw_ref[...] = jnp.mean(esk_ref[...], axis=1)
    cw_ref[...] = jnp.mean(ecw_ref[...], axis=1)


def _run_tc_means(esk, ecw):
    nb = 8
    blk = B // nb
    f32 = jnp.float32
    return pl.pallas_call(
        _tc_mean_kernel,
        grid=(nb,),
        in_specs=[pl.BlockSpec((blk, 20, EMB), lambda i: (i, 0, 0)),
                  pl.BlockSpec((blk, 20, EMB), lambda i: (i, 0, 0))],
        out_specs=[pl.BlockSpec((blk, EMB), lambda i: (i, 0)),
                   pl.BlockSpec((blk, EMB), lambda i: (i, 0))],
        out_shape=[jax.ShapeDtypeStruct((B, EMB), f32),
                   jax.ShapeDtypeStruct((B, EMB), f32)],
    )(esk, ecw)


def _tc_tower_kernel(avg_ref, kw_ref, cw_ref, lev_ref, loc_ref, misc_ref,
                     bn1g_ref, bn1b_ref, wf_ref, bf_ref, bn2g_ref, bn2b_ref,
                     w1_ref, b1_ref, w2_ref, b2_ref, w3_ref, b3_ref,
                     w4_ref, b4_ref, out_ref):
    uf = jnp.concatenate([avg_ref[...], kw_ref[...], cw_ref[...],
                          lev_ref[...], loc_ref[...], misc_ref[...]], axis=1)

    def bn(x, g, b):
        mu = jnp.mean(x, axis=0)
        xc = x - mu
        var = jnp.mean(xc * xc, axis=0)
        return g * xc / jnp.sqrt(var + 1e-5) + b

    h = bn(uf, bn1g_ref[...], bn1b_ref[...])
    h = jnp.maximum(jnp.dot(h, wf_ref[...]) + bf_ref[...], 0.0)
    h = bn(h, bn2g_ref[...], bn2b_ref[...])
    h = jnp.maximum(jnp.dot(h, w1_ref[...]) + b1_ref[...], 0.0)
    h = jnp.maximum(jnp.dot(h, w2_ref[...]) + b2_ref[...], 0.0)
    h = jnp.maximum(jnp.dot(h, w3_ref[...]) + b3_ref[...], 0.0)
    h = jnp.maximum(jnp.dot(h, w4_ref[...]) + b4_ref[...], 0.0)
    out_ref[...] = h


def _run_tc_tower(avg_video, kw_mean, cw_mean, lev_e, loc_e, misc,
                  bn1_g, bn1_b, WfT, b_fus, bn2_g, bn2_b,
                  W1T, b1, W2T, b2, W3T, b3, W4T, b4):
    f32 = jnp.float32
    args = (avg_video, kw_mean, cw_mean, lev_e, loc_e, misc,
            bn1_g, bn1_b, WfT, b_fus, bn2_g, bn2_b,
            W1T, b1, W2T, b2, W3T, b3, W4T, b4)
    return pl.pallas_call(
        _tc_tower_kernel,
        out_shape=jax.ShapeDtypeStruct((B, 64), f32),
    )(*args)


def kernel(watched_videos, embed_searched_keywords, embed_collect_words,
           level, location, gender, age, platform, example_age, samples,
           table_video, table_sample, table_location,
           bn1_g, bn1_b, W_fus, b_fus, bn2_g, bn2_b,
           W1, b1, W2, b2, W3, b3, W4, b4):
    i32 = jnp.int32
    wv_pad = watched_videos.astype(i32).reshape(B * HIST)
    lev = level.astype(i32).reshape(B)
    loc = location.astype(i32).reshape(B)
    samp = samples.astype(i32)

    avg_video, lev_e, loc_e, samp_e = _run_sc_gathers(
        wv_pad, lev, loc, samp, table_video, table_location, table_sample)

    kw_mean, cw_mean = _run_tc_means(embed_searched_keywords, embed_collect_words)

    misc = jnp.concatenate(
        [gender, age, platform, example_age, example_age * example_age], axis=1)

    h = _run_tc_tower(avg_video, kw_mean, cw_mean, lev_e, loc_e, misc,
                      bn1_g, bn1_b, W_fus.T, b_fus, bn2_g, bn2_b,
                      W1.T, b1, W2.T, b2, W3.T, b3, W4.T, b4)
    return (h, samp_e)
